# baseline (speedup 1.0000x reference)
import functools

import jax
import jax.numpy as jnp
from jax import lax
from jax.experimental import pallas as pl
from jax.experimental.pallas import tpu as pltpu
from jax.experimental.pallas import tpu_sc as plsc

_N = 10000
_E = 320000
_D = 128
_H = 128
_C = 10
_G = 128

_NC = 2
_NS = 16
_NW = _NC * _NS
_K = 128
_ECH = 80
_EP = _NW * _ECH * _K
_ECH0 = 62
_ECH1 = 2 * _ECH - _ECH0
_NP = 10240
_RPT = _NP // _NS

_RB = 1000
_NB = _N // _RB

_sc_mesh = functools.partial(plsc.VectorSubcoreMesh,
                             core_axis_name="c", subcore_axis_name="s")


@functools.partial(
    pl.kernel,
    mesh=_sc_mesh(),
    out_type=jax.ShapeDtypeStruct((_NC * _NP, _D), jnp.float32),
    scratch_types=[
        pltpu.VMEM((_K,), jnp.int32),
        pltpu.VMEM((_K,), jnp.int32),
        pltpu.VMEM((_K,), jnp.int32),
        pltpu.VMEM((_K,), jnp.int32),
        pltpu.VMEM((_K, _D), jnp.float32),
        pltpu.VMEM((_K, _D), jnp.float32),
        pltpu.VMEM_SHARED((_NP, _D), jnp.float32),
        pltpu.SemaphoreType.DMA,
        pltpu.SemaphoreType.DMA,
    ],
)
def _sc_agg(y, srcp, dstp, zin, out, s0, s1, d0, d1, r0, r1, acc, sem0, sem1):
    cid = lax.axis_index("c")
    sid = lax.axis_index("s")
    base = sid * _RPT

    pltpu.sync_copy(zin, r0)
    for t in range(_RPT // _K):
        pltpu.sync_copy(r0, acc.at[pl.ds(base + t * _K, _K)])
    plsc.subcore_barrier()

    def _sb_loop(woff, nch):
        def _body(c, carry):
            off = woff + c * _K
            pltpu.sync_copy(srcp.at[pl.ds(off, _K)], s0)
            pltpu.sync_copy(dstp.at[pl.ds(off, _K)], d0)
            pltpu.async_copy(y.at[s0], r0, sem0).wait()
            pltpu.sync_copy(r0, acc.at[d0], add=True)
            return carry
        lax.fori_loop(0, nch, _body, 0)

    @pl.when(cid == 0)
    def _core0():
        _sb_loop(sid * _ECH0 * _K, _ECH0)

    @pl.when(cid == 1)
    def _core1():
        _sb_loop((_NS * _ECH0 + sid * _ECH1) * _K, _ECH1)

    plsc.subcore_barrier()

    pltpu.sync_copy(acc.at[pl.ds(base, _RPT)],
                    out.at[pl.ds(cid * _NP + base, _RPT)])


@functools.partial(
    pl.kernel,
    mesh=_sc_mesh(),
    out_type=jax.ShapeDtypeStruct((_NC * _NP, _D), jnp.float32),
    scratch_types=[
        pltpu.VMEM((_K,), jnp.int32),
        pltpu.VMEM((_K, _D), jnp.float32),
        pltpu.VMEM_SHARED((_NP, _D), jnp.float32),
    ],
)
def _sc_deg(dstp, zin, oin, out, didx, ones, dacc):
    cid = lax.axis_index("c")
    sid = lax.axis_index("s")
    wid = cid * _NS + sid
    base = sid * _RPT

    pltpu.sync_copy(zin, ones)
    for t in range(_RPT // _K):
        pltpu.sync_copy(ones, dacc.at[pl.ds(base + t * _K, _K)])
    pltpu.sync_copy(oin, ones)
    plsc.subcore_barrier()

    def _body(c, carry):
        off = (wid * _ECH + c) * _K
        pltpu.sync_copy(dstp.at[pl.ds(off, _K)], didx)
        pltpu.sync_copy(ones, dacc.at[didx], add=True)
        return carry
    lax.fori_loop(0, _ECH, _body, 0)
    plsc.subcore_barrier()

    pltpu.sync_copy(dacc.at[pl.ds(base, _RPT)],
                    out.at[pl.ds(cid * _NP + base, _RPT)])


def _tc_layer(acc2, deg2, h, Wl, bl, Wr, relu):
    def body(acc_ref, deg_ref, h_ref, wl_ref, b_ref, wr_ref, o_ref):
        deg = deg_ref[0, :, 0:1] + deg_ref[1, :, 0:1]
        agg = (acc_ref[0] + acc_ref[1]) / jnp.maximum(deg, 1.0)
        o = (jnp.dot(agg, wl_ref[...], preferred_element_type=jnp.float32)
             + jnp.dot(h_ref[...], wr_ref[...], preferred_element_type=jnp.float32)
             + b_ref[...])
        if relu:
            o = jnp.maximum(o, 0.0)
        o_ref[...] = o

    return pl.pallas_call(
        body,
        grid=(_NB,),
        in_specs=[
            pl.BlockSpec((_NC, _RB, _D), lambda i: (0, i, 0)),
            pl.BlockSpec((_NC, _RB, _D), lambda i: (0, i, 0)),
            pl.BlockSpec((_RB, _D), lambda i: (i, 0)),
            pl.BlockSpec((_D, _H), lambda i: (0, 0)),
            pl.BlockSpec((1, _H), lambda i: (0, 0)),
            pl.BlockSpec((_D, _H), lambda i: (0, 0)),
        ],
        out_specs=pl.BlockSpec((_RB, _H), lambda i: (i, 0)),
        out_shape=jax.ShapeDtypeStruct((_N, _H), jnp.float32),
    )(acc2, deg2, h, Wl, bl.reshape(1, _H), Wr)


def _tc_pool(h3, batch3, Wlin, blin):
    def body(h_ref, b_ref, wl_ref, bl_ref, o_ref, psum, csum):
        i = pl.program_id(0)

        @pl.when(i == 0)
        def _init():
            psum[...] = jnp.zeros_like(psum)
            csum[...] = jnp.zeros_like(csum)

        ids = b_ref[0, 0, :]
        onehot = (ids[:, None]
                  == lax.broadcasted_iota(jnp.int32, (_RB, _G), 1)).astype(jnp.float32)
        psum[...] += lax.dot_general(
            onehot, h_ref[...], (((0,), (0,)), ((), ())),
            preferred_element_type=jnp.float32)
        csum[...] += lax.dot_general(
            onehot, jnp.ones((_RB, 1), jnp.float32), (((0,), (0,)), ((), ())),
            preferred_element_type=jnp.float32)

        @pl.when(i == _NB - 1)
        def _fin():
            pooled = psum[...] / jnp.maximum(csum[...], 1.0)
            o_ref[...] = (jnp.dot(pooled, wl_ref[...],
                                  preferred_element_type=jnp.float32)
                          + bl_ref[...])

    return pl.pallas_call(
        body,
        grid=(_NB,),
        in_specs=[
            pl.BlockSpec((_RB, _H), lambda i: (i, 0)),
            pl.BlockSpec((1, 1, _RB), lambda i: (i, 0, 0)),
            pl.BlockSpec((_H, _C), lambda i: (0, 0)),
            pl.BlockSpec((1, _C), lambda i: (0, 0)),
        ],
        out_specs=pl.BlockSpec((_G, _C), lambda i: (0, 0)),
        out_shape=jax.ShapeDtypeStruct((_G, _C), jnp.float32),
        scratch_shapes=[
            pltpu.VMEM((_G, _H), jnp.float32),
            pltpu.VMEM((_G, 1), jnp.float32),
        ],
    )(h3, batch3, Wlin, blin.reshape(1, _C))


def kernel(x, edge_index, batch, W1l, b1l, W1r, W2l, b2l, W2r, W3l, b3l, W3r,
           Wlin, blin):
    ei = edge_index.astype(jnp.int32)
    pad = _EP - _E
    srcp = jnp.concatenate([ei[0], jnp.zeros((pad + 4 * _K,), jnp.int32)])
    dstp = jnp.concatenate(
        [ei[1], _N + (jnp.arange(pad, dtype=jnp.int32) % (_NP - _N))])
    batch3 = batch.astype(jnp.int32).reshape(_NB, 1, _RB)

    zin = jnp.zeros((_K, _D), jnp.float32)
    oin = jnp.ones((_K, _D), jnp.float32)

    deg = _sc_deg(dstp, zin, oin).reshape(_NC, _NP, _D)
    agg1 = _sc_agg(x, srcp, dstp, zin).reshape(_NC, _NP, _D)
    h1 = _tc_layer(agg1, deg, x, W1l, b1l, W1r, relu=True)
    agg2 = _sc_agg(h1, srcp, dstp, zin).reshape(_NC, _NP, _D)
    h2 = _tc_layer(agg2, deg, h1, W2l, b2l, W2r, relu=True)
    agg3 = _sc_agg(h2, srcp, dstp, zin).reshape(_NC, _NP, _D)
    h3 = _tc_layer(agg3, deg, h2, W3l, b3l, W3r, relu=False)
    return _tc_pool(h3, batch3, Wlin, blin)


# speedup vs baseline: 1.6110x; 1.6110x over previous
import functools

import jax
import jax.numpy as jnp
from jax import lax
from jax.experimental import pallas as pl
from jax.experimental.pallas import tpu as pltpu
from jax.experimental.pallas import tpu_sc as plsc

_N = 10000
_E = 320000
_D = 128
_H = 128
_C = 10
_G = 128

_NC = 2
_NS = 16
_NW = _NC * _NS
_K = 128
_ECH = 79
_EP = _NW * _ECH * _K
_NP = 10240
_RPT = _NP // _NS

_RB = 1000
_NB = _N // _RB

_sc_mesh = functools.partial(plsc.VectorSubcoreMesh,
                             core_axis_name="c", subcore_axis_name="s")


@functools.partial(
    pl.kernel,
    mesh=_sc_mesh(),
    out_type=jax.ShapeDtypeStruct((_NC * _NP, _D), jnp.float32),
    scratch_types=[
        pltpu.VMEM((_K,), jnp.int32),
        pltpu.VMEM((_K,), jnp.int32),
        pltpu.VMEM((_K, _D), jnp.float32),
        pltpu.VMEM_SHARED((_NP, _D), jnp.float32),
        pltpu.SemaphoreType.DMA,
    ],
)
def _sc_agg(y, srcp, dstp, zin, out, sidx, didx, rows, acc, sem):
    cid = lax.axis_index("c")
    sid = lax.axis_index("s")
    wid = cid * _NS + sid
    base = sid * _RPT

    pltpu.sync_copy(zin, rows)
    for t in range(_RPT // _K):
        pltpu.sync_copy(rows, acc.at[pl.ds(base + t * _K, _K)])
    plsc.subcore_barrier()

    def _body(c, carry):
        off = (wid * _ECH + c) * _K
        pltpu.sync_copy(srcp.at[pl.ds(off, _K)], sidx)
        pltpu.sync_copy(dstp.at[pl.ds(off, _K)], didx)
        pltpu.async_copy(y.at[sidx], rows, sem).wait()
        pltpu.sync_copy(rows, acc.at[didx], add=True)
        return carry
    lax.fori_loop(0, _ECH, _body, 0)
    plsc.subcore_barrier()

    pltpu.sync_copy(acc.at[pl.ds(base, _RPT)],
                    out.at[pl.ds(cid * _NP + base, _RPT)])


@functools.partial(
    pl.kernel,
    mesh=_sc_mesh(),
    out_type=jax.ShapeDtypeStruct((_NC * _NP, _D), jnp.float32),
    scratch_types=[
        pltpu.VMEM((_K,), jnp.int32),
        pltpu.VMEM((_K, _D), jnp.float32),
        pltpu.VMEM_SHARED((_NP, _D), jnp.float32),
    ],
)
def _sc_deg(dstp, zin, oin, out, didx, ones, dacc):
    cid = lax.axis_index("c")
    sid = lax.axis_index("s")
    wid = cid * _NS + sid
    base = sid * _RPT

    pltpu.sync_copy(zin, ones)
    for t in range(_RPT // _K):
        pltpu.sync_copy(ones, dacc.at[pl.ds(base + t * _K, _K)])
    pltpu.sync_copy(oin, ones)
    plsc.subcore_barrier()

    def _body(c, carry):
        off = (wid * _ECH + c) * _K
        pltpu.sync_copy(dstp.at[pl.ds(off, _K)], didx)
        pltpu.sync_copy(ones, dacc.at[didx], add=True)
        return carry
    lax.fori_loop(0, _ECH, _body, 0)
    plsc.subcore_barrier()

    pltpu.sync_copy(dacc.at[pl.ds(base, _RPT)],
                    out.at[pl.ds(cid * _NP + base, _RPT)])


def _tc_layer(acc2, deg2, h, Wl, bl, Wr, relu):
    def body(acc_ref, deg_ref, h_ref, wl_ref, b_ref, wr_ref, o_ref):
        deg = deg_ref[0, :, 0:1] + deg_ref[1, :, 0:1]
        agg = (acc_ref[0] + acc_ref[1]) / jnp.maximum(deg, 1.0)
        o = (jnp.dot(agg, wl_ref[...], preferred_element_type=jnp.float32)
             + jnp.dot(h_ref[...], wr_ref[...], preferred_element_type=jnp.float32)
             + b_ref[...])
        if relu:
            o = jnp.maximum(o, 0.0)
        o_ref[...] = o

    return pl.pallas_call(
        body,
        grid=(_NB,),
        in_specs=[
            pl.BlockSpec((_NC, _RB, _D), lambda i: (0, i, 0)),
            pl.BlockSpec((_NC, _RB, _D), lambda i: (0, i, 0)),
            pl.BlockSpec((_RB, _D), lambda i: (i, 0)),
            pl.BlockSpec((_D, _H), lambda i: (0, 0)),
            pl.BlockSpec((1, _H), lambda i: (0, 0)),
            pl.BlockSpec((_D, _H), lambda i: (0, 0)),
        ],
        out_specs=pl.BlockSpec((_RB, _H), lambda i: (i, 0)),
        out_shape=jax.ShapeDtypeStruct((_N, _H), jnp.float32),
    )(acc2, deg2, h, Wl, bl.reshape(1, _H), Wr)


def _tc_pool(h3, batch3, Wlin, blin):
    def body(h_ref, b_ref, wl_ref, bl_ref, o_ref, psum, csum):
        i = pl.program_id(0)

        @pl.when(i == 0)
        def _init():
            psum[...] = jnp.zeros_like(psum)
            csum[...] = jnp.zeros_like(csum)

        ids = b_ref[0, 0, :]
        onehot = (ids[:, None]
                  == lax.broadcasted_iota(jnp.int32, (_RB, _G), 1)).astype(jnp.float32)
        psum[...] += lax.dot_general(
            onehot, h_ref[...], (((0,), (0,)), ((), ())),
            preferred_element_type=jnp.float32)
        csum[...] += lax.dot_general(
            onehot, jnp.ones((_RB, 1), jnp.float32), (((0,), (0,)), ((), ())),
            preferred_element_type=jnp.float32)

        @pl.when(i == _NB - 1)
        def _fin():
            pooled = psum[...] / jnp.maximum(csum[...], 1.0)
            o_ref[...] = (jnp.dot(pooled, wl_ref[...],
                                  preferred_element_type=jnp.float32)
                          + bl_ref[...])

    return pl.pallas_call(
        body,
        grid=(_NB,),
        in_specs=[
            pl.BlockSpec((_RB, _H), lambda i: (i, 0)),
            pl.BlockSpec((1, 1, _RB), lambda i: (i, 0, 0)),
            pl.BlockSpec((_H, _C), lambda i: (0, 0)),
            pl.BlockSpec((1, _C), lambda i: (0, 0)),
        ],
        out_specs=pl.BlockSpec((_G, _C), lambda i: (0, 0)),
        out_shape=jax.ShapeDtypeStruct((_G, _C), jnp.float32),
        scratch_shapes=[
            pltpu.VMEM((_G, _H), jnp.float32),
            pltpu.VMEM((_G, 1), jnp.float32),
        ],
    )(h3, batch3, Wlin, blin.reshape(1, _C))


def kernel(x, edge_index, batch, W1l, b1l, W1r, W2l, b2l, W2r, W3l, b3l, W3r,
           Wlin, blin):
    ei = edge_index.astype(jnp.int32)
    pad = _EP - _E
    srcp = jnp.concatenate([ei[0], jnp.zeros((pad,), jnp.int32)])
    dstp = jnp.concatenate([ei[1], jnp.full((pad,), _N, jnp.int32)])
    batch3 = batch.astype(jnp.int32).reshape(_NB, 1, _RB)

    zin = jnp.zeros((_K, _D), jnp.float32)
    oin = jnp.ones((_K, _D), jnp.float32)

    deg = _sc_deg(dstp, zin, oin).reshape(_NC, _NP, _D)
    agg1 = _sc_agg(x, srcp, dstp, zin).reshape(_NC, _NP, _D)
    h1 = _tc_layer(agg1, deg, x, W1l, b1l, W1r, relu=True)
    agg2 = _sc_agg(h1, srcp, dstp, zin).reshape(_NC, _NP, _D)
    h2 = _tc_layer(agg2, deg, h1, W2l, b2l, W2r, relu=True)
    agg3 = _sc_agg(h2, srcp, dstp, zin).reshape(_NC, _NP, _D)
    h3 = _tc_layer(agg3, deg, h2, W3l, b3l, W3r, relu=False)
    return _tc_pool(h3, batch3, Wlin, blin)


# speedup vs baseline: 1.6533x; 1.0262x over previous
import functools

import jax
import jax.numpy as jnp
from jax import lax
from jax.experimental import pallas as pl
from jax.experimental.pallas import tpu as pltpu
from jax.experimental.pallas import tpu_sc as plsc

_N = 10000
_E = 320000
_D = 128
_H = 128
_C = 10
_G = 128

_NC = 2
_NS = 16
_NW = _NC * _NS
_K = 128
_ECH = 79
_EP = _NW * _ECH * _K
_NP = 10240
_RPT = _NP // _NS

_RB = 1000
_NB = _N // _RB

_sc_mesh = functools.partial(plsc.VectorSubcoreMesh,
                             core_axis_name="c", subcore_axis_name="s")


@functools.partial(
    pl.kernel,
    mesh=_sc_mesh(),
    out_type=jax.ShapeDtypeStruct((_NC * _NP, _D), jnp.float32),
    scratch_types=[
        pltpu.VMEM((2, _K), jnp.int32),
        pltpu.VMEM((_K, _D), jnp.float32),
        pltpu.VMEM_SHARED((_NP, _D), jnp.float32),
        pltpu.SemaphoreType.DMA,
    ],
)
def _sc_agg(y, eidx, zin, out, idx, rows, acc, sem):
    cid = lax.axis_index("c")
    sid = lax.axis_index("s")
    wid = cid * _NS + sid
    base = sid * _RPT

    pltpu.sync_copy(zin, rows)
    for t in range(_RPT // _K):
        pltpu.sync_copy(rows, acc.at[pl.ds(base + t * _K, _K)])
    plsc.subcore_barrier()

    def _body(c, carry):
        pltpu.sync_copy(eidx.at[wid * _ECH + c], idx)
        pltpu.async_copy(y.at[idx.at[0]], rows, sem).wait()
        pltpu.sync_copy(rows, acc.at[idx.at[1]], add=True)
        return carry
    lax.fori_loop(0, _ECH, _body, 0)
    plsc.subcore_barrier()

    pltpu.sync_copy(acc.at[pl.ds(base, _RPT)],
                    out.at[pl.ds(cid * _NP + base, _RPT)])


@functools.partial(
    pl.kernel,
    mesh=_sc_mesh(),
    out_type=jax.ShapeDtypeStruct((_NC * _NP, _D), jnp.float32),
    scratch_types=[
        pltpu.VMEM((_K,), jnp.int32),
        pltpu.VMEM((_K, _D), jnp.float32),
        pltpu.VMEM_SHARED((_NP, _D), jnp.float32),
    ],
)
def _sc_deg(dstp, zin, oin, out, didx, ones, dacc):
    cid = lax.axis_index("c")
    sid = lax.axis_index("s")
    wid = cid * _NS + sid
    base = sid * _RPT

    pltpu.sync_copy(zin, ones)
    for t in range(_RPT // _K):
        pltpu.sync_copy(ones, dacc.at[pl.ds(base + t * _K, _K)])
    pltpu.sync_copy(oin, ones)
    plsc.subcore_barrier()

    def _body(c, carry):
        off = (wid * _ECH + c) * _K
        pltpu.sync_copy(dstp.at[pl.ds(off, _K)], didx)
        pltpu.sync_copy(ones, dacc.at[didx], add=True)
        return carry
    lax.fori_loop(0, _ECH, _body, 0)
    plsc.subcore_barrier()

    pltpu.sync_copy(dacc.at[pl.ds(base, _RPT)],
                    out.at[pl.ds(cid * _NP + base, _RPT)])


def _tc_layer(acc2, deg2, h, Wl, bl, Wr, relu):
    def body(acc_ref, deg_ref, h_ref, wl_ref, b_ref, wr_ref, o_ref):
        deg = deg_ref[0, :, 0:1] + deg_ref[1, :, 0:1]
        agg = (acc_ref[0] + acc_ref[1]) / jnp.maximum(deg, 1.0)
        o = (jnp.dot(agg, wl_ref[...], preferred_element_type=jnp.float32)
             + jnp.dot(h_ref[...], wr_ref[...], preferred_element_type=jnp.float32)
             + b_ref[...])
        if relu:
            o = jnp.maximum(o, 0.0)
        o_ref[...] = o

    return pl.pallas_call(
        body,
        grid=(_NB,),
        in_specs=[
            pl.BlockSpec((_NC, _RB, _D), lambda i: (0, i, 0)),
            pl.BlockSpec((_NC, _RB, _D), lambda i: (0, i, 0)),
            pl.BlockSpec((_RB, _D), lambda i: (i, 0)),
            pl.BlockSpec((_D, _H), lambda i: (0, 0)),
            pl.BlockSpec((1, _H), lambda i: (0, 0)),
            pl.BlockSpec((_D, _H), lambda i: (0, 0)),
        ],
        out_specs=pl.BlockSpec((_RB, _H), lambda i: (i, 0)),
        out_shape=jax.ShapeDtypeStruct((_N, _H), jnp.float32),
    )(acc2, deg2, h, Wl, bl.reshape(1, _H), Wr)


def _tc_pool(h3, batch3, Wlin, blin):
    def body(h_ref, b_ref, wl_ref, bl_ref, o_ref, psum, csum):
        i = pl.program_id(0)

        @pl.when(i == 0)
        def _init():
            psum[...] = jnp.zeros_like(psum)
            csum[...] = jnp.zeros_like(csum)

        ids = b_ref[0, 0, :]
        onehot = (ids[:, None]
                  == lax.broadcasted_iota(jnp.int32, (_RB, _G), 1)).astype(jnp.float32)
        psum[...] += lax.dot_general(
            onehot, h_ref[...], (((0,), (0,)), ((), ())),
            preferred_element_type=jnp.float32)
        csum[...] += lax.dot_general(
            onehot, jnp.ones((_RB, 1), jnp.float32), (((0,), (0,)), ((), ())),
            preferred_element_type=jnp.float32)

        @pl.when(i == _NB - 1)
        def _fin():
            pooled = psum[...] / jnp.maximum(csum[...], 1.0)
            o_ref[...] = (jnp.dot(pooled, wl_ref[...],
                                  preferred_element_type=jnp.float32)
                          + bl_ref[...])

    return pl.pallas_call(
        body,
        grid=(_NB,),
        in_specs=[
            pl.BlockSpec((_RB, _H), lambda i: (i, 0)),
            pl.BlockSpec((1, 1, _RB), lambda i: (i, 0, 0)),
            pl.BlockSpec((_H, _C), lambda i: (0, 0)),
            pl.BlockSpec((1, _C), lambda i: (0, 0)),
        ],
        out_specs=pl.BlockSpec((_G, _C), lambda i: (0, 0)),
        out_shape=jax.ShapeDtypeStruct((_G, _C), jnp.float32),
        scratch_shapes=[
            pltpu.VMEM((_G, _H), jnp.float32),
            pltpu.VMEM((_G, 1), jnp.float32),
        ],
    )(h3, batch3, Wlin, blin.reshape(1, _C))


def kernel(x, edge_index, batch, W1l, b1l, W1r, W2l, b2l, W2r, W3l, b3l, W3r,
           Wlin, blin):
    ei = edge_index.astype(jnp.int32)
    pad = _EP - _E
    srcp = jnp.concatenate([ei[0], jnp.zeros((pad,), jnp.int32)])
    dstp = jnp.concatenate([ei[1], jnp.full((pad,), _N, jnp.int32)])
    batch3 = batch.astype(jnp.int32).reshape(_NB, 1, _RB)

    zin = jnp.zeros((_K, _D), jnp.float32)
    oin = jnp.ones((_K, _D), jnp.float32)

    eidx = jnp.stack(
        [srcp.reshape(-1, _K), dstp.reshape(-1, _K)], axis=1)

    deg = _sc_deg(dstp, zin, oin).reshape(_NC, _NP, _D)
    agg1 = _sc_agg(x, eidx, zin).reshape(_NC, _NP, _D)
    h1 = _tc_layer(agg1, deg, x, W1l, b1l, W1r, relu=True)
    agg2 = _sc_agg(h1, eidx, zin).reshape(_NC, _NP, _D)
    h2 = _tc_layer(agg2, deg, h1, W2l, b2l, W2r, relu=True)
    agg3 = _sc_agg(h2, eidx, zin).reshape(_NC, _NP, _D)
    h3 = _tc_layer(agg3, deg, h2, W3l, b3l, W3r, relu=False)
    return _tc_pool(h3, batch3, Wlin, blin)
